# baseline (speedup 1.0000x reference)
import functools

import jax
import jax.numpy as jnp
from jax import lax
from jax.experimental import pallas as pl
from jax.experimental.pallas import tpu as pltpu
from jax.experimental.pallas import tpu_sc as plsc

N = 10000
E = 320000
D = 128
NC = 2
NS = 16
NW = NC * NS
EPW = E // NW
K = 128
NCHUNK = 80
EPW_PAD = NCHUNK * K
ACC_ROWS = 10016
RPT = 624
REM = N - RPT * NS


def _sc_partial(data, se, de, zeros):
    mesh = plsc.VectorSubcoreMesh(
        core_axis_name="c", subcore_axis_name="s", num_cores=NC
    )

    @functools.partial(
        pl.kernel,
        out_type=jax.ShapeDtypeStruct((NC, N, D), jnp.float32),
        mesh=mesh,
        scratch_types=[pltpu.VMEM_SHARED((ACC_ROWS, D), jnp.float32)]
        + [pltpu.VMEM((K,), jnp.int32) for _ in range(4)]
        + [pltpu.VMEM((K, D), jnp.float32) for _ in range(2)]
        + [pltpu.SemaphoreType.DMA for _ in range(4)],
    )
    def k(data_hbm, se_hbm, de_hbm, zero_hbm, out_hbm, acc,
          src0, src1, dst0, dst1, rows0, rows1, isem0, isem1, gsem0, gsem1):
        srcb = (src0, src1)
        dstb = (dst0, dst1)
        rows = (rows0, rows1)
        isem = (isem0, isem1)
        gsem = (gsem0, gsem1)
        c = lax.axis_index("c")
        s = lax.axis_index("s")
        wid = s * NC + c

        pltpu.sync_copy(
            zero_hbm.at[pl.ds(s * RPT, RPT)], acc.at[pl.ds(s * RPT, RPT)]
        )

        @pl.when(s == 0)
        def _zero_rem():
            pltpu.sync_copy(
                zero_hbm.at[pl.ds(RPT * NS, REM)], acc.at[pl.ds(RPT * NS, REM)]
            )

        plsc.subcore_barrier()

        base0 = wid * EPW_PAD

        def start_idx(g, b):
            pltpu.async_copy(se_hbm.at[pl.ds(base0 + g * K, K)], srcb[b], isem[b])
            pltpu.async_copy(de_hbm.at[pl.ds(base0 + g * K, K)], dstb[b], isem[b])

        def wait_idx(g, b):
            pltpu.make_async_copy(
                se_hbm.at[pl.ds(base0 + g * K, K)], srcb[b], isem[b]
            ).wait()
            pltpu.make_async_copy(
                de_hbm.at[pl.ds(base0 + g * K, K)], dstb[b], isem[b]
            ).wait()

        def start_gather(b):
            pltpu.async_copy(data_hbm.at[srcb[b]], rows[b], gsem[b])

        def wait_gather(b):
            pltpu.make_async_copy(data_hbm.at[srcb[b]], rows[b], gsem[b]).wait()

        def scatter(b):
            pltpu.sync_copy(rows[b], acc.at[dstb[b]], add=True)

        start_idx(0, 0)
        start_idx(1, 1)
        wait_idx(0, 0)
        start_gather(0)

        @pl.loop(0, NCHUNK - 2, step=2)
        def _grp(g0):
            for b in range(2):
                g = g0 + b
                b2 = 1 - b
                wait_gather(b)
                wait_idx(g + 1, b2)
                start_gather(b2)
                scatter(b)

                @pl.when(g + 2 < NCHUNK)
                def _prefetch_idx():
                    start_idx(g + 2, b)

        wait_gather(0)
        wait_idx(NCHUNK - 1, 1)
        start_gather(1)
        scatter(0)
        wait_gather(1)
        scatter(1)

        plsc.subcore_barrier()
        pltpu.sync_copy(
            acc.at[pl.ds(s * RPT, RPT)], out_hbm.at[c, pl.ds(s * RPT, RPT)]
        )

        @pl.when(s == 0)
        def _flush_rem():
            pltpu.sync_copy(
                acc.at[pl.ds(RPT * NS, REM)], out_hbm.at[c, pl.ds(RPT * NS, REM)]
            )

    return k(data, se, de, zeros)


def _combine(partial):
    def body(p_ref, o_ref):
        o_ref[...] = p_ref[0] + p_ref[1]

    return pl.pallas_call(
        body,
        out_shape=jax.ShapeDtypeStruct((N, D), jnp.float32),
        grid=(10,),
        in_specs=[pl.BlockSpec((2, 1000, D), lambda i: (0, i, 0))],
        out_specs=pl.BlockSpec((1000, D), lambda i: (i, 0)),
    )(partial)


@jax.jit
def kernel(data, edge_index):
    pad = EPW_PAD - EPW
    src = edge_index[0].reshape(NW, EPW)
    dst = edge_index[1].reshape(NW, EPW)
    se = jnp.pad(src, ((0, 0), (0, pad))).reshape(NW * EPW_PAD)
    dump = (N + jnp.arange(NW, dtype=jnp.int32) // NC)[:, None]
    de = jnp.concatenate(
        [dst, jnp.broadcast_to(dump, (NW, pad))], axis=1
    ).reshape(NW * EPW_PAD)
    zeros = jnp.zeros((N, D), jnp.float32)
    partial = _sc_partial(data, se, de, zeros)
    return _combine(partial)


# speedup vs baseline: 2.6127x; 2.6127x over previous
import functools

import jax
import jax.numpy as jnp
from jax import lax
from jax.experimental import pallas as pl
from jax.experimental.pallas import tpu as pltpu
from jax.experimental.pallas import tpu_sc as plsc

N = 10000
E = 320000
D = 128
NC = 2
NS = 16
NW = NC * NS
EPW = E // NW
K = 80
NCHUNK = EPW // K
NB = 3
RPT = 624
REM = N - RPT * NS


def _sc_partial(data, se, de, zeros):
    mesh = plsc.VectorSubcoreMesh(
        core_axis_name="c", subcore_axis_name="s", num_cores=NC
    )

    @functools.partial(
        pl.kernel,
        out_type=jax.ShapeDtypeStruct((NC, N, D), jnp.float32),
        mesh=mesh,
        scratch_types=[pltpu.VMEM_SHARED((N, D), jnp.float32)]
        + [pltpu.VMEM((K,), jnp.int32) for _ in range(2 * NB)]
        + [pltpu.VMEM((K, D), jnp.float32) for _ in range(NB)]
        + [pltpu.SemaphoreType.DMA for _ in range(2 * NB)],
    )
    def k(data_hbm, se_hbm, de_hbm, zero_hbm, out_hbm, acc, *scr):
        srcb = scr[0:NB]
        dstb = scr[NB:2 * NB]
        rows = scr[2 * NB:3 * NB]
        isem = scr[3 * NB:4 * NB]
        gsem = scr[4 * NB:5 * NB]
        c = lax.axis_index("c")
        s = lax.axis_index("s")
        wid = s * NC + c

        pltpu.sync_copy(
            zero_hbm.at[pl.ds(s * RPT, RPT)], acc.at[pl.ds(s * RPT, RPT)]
        )

        @pl.when(s == 0)
        def _zero_rem():
            pltpu.sync_copy(
                zero_hbm.at[pl.ds(RPT * NS, REM)], acc.at[pl.ds(RPT * NS, REM)]
            )

        plsc.subcore_barrier()

        base0 = wid * EPW

        def start_idx(g, b):
            pltpu.async_copy(se_hbm.at[pl.ds(base0 + g * K, K)], srcb[b], isem[b])
            pltpu.async_copy(de_hbm.at[pl.ds(base0 + g * K, K)], dstb[b], isem[b])

        def wait_idx(g, b):
            pltpu.make_async_copy(
                se_hbm.at[pl.ds(base0 + g * K, K)], srcb[b], isem[b]
            ).wait()
            pltpu.make_async_copy(
                de_hbm.at[pl.ds(base0 + g * K, K)], dstb[b], isem[b]
            ).wait()

        def start_gather(b):
            pltpu.async_copy(data_hbm.at[srcb[b]], rows[b], gsem[b])

        def wait_gather(b):
            pltpu.make_async_copy(data_hbm.at[srcb[b]], rows[b], gsem[b]).wait()

        def scatter(b):
            pltpu.sync_copy(rows[b], acc.at[dstb[b]], add=True)

        for b in range(NB):
            start_idx(b, b)
        for b in range(2):
            wait_idx(b, b)
            start_gather(b)

        @pl.loop(0, NCHUNK - 2, step=NB)
        def _grp(g0):
            for b in range(NB):
                g = g0 + b
                b2 = (b + 2) % NB
                wait_gather(b)
                wait_idx(g + 2, b2)
                start_gather(b2)
                scatter(b)

                @pl.when(g + NB < NCHUNK)
                def _prefetch_idx():
                    start_idx(g + NB, b)

        wait_gather(0)
        scatter(0)
        wait_gather(1)
        scatter(1)

        plsc.subcore_barrier()
        pltpu.sync_copy(
            acc.at[pl.ds(s * RPT, RPT)], out_hbm.at[c, pl.ds(s * RPT, RPT)]
        )

        @pl.when(s == 0)
        def _flush_rem():
            pltpu.sync_copy(
                acc.at[pl.ds(RPT * NS, REM)], out_hbm.at[c, pl.ds(RPT * NS, REM)]
            )

    return k(data, se, de, zeros)


def _combine(partial):
    def body(p_ref, o_ref):
        o_ref[...] = p_ref[0] + p_ref[1]

    return pl.pallas_call(
        body,
        out_shape=jax.ShapeDtypeStruct((N, D), jnp.float32),
        grid=(10,),
        in_specs=[pl.BlockSpec((2, 1000, D), lambda i: (0, i, 0))],
        out_specs=pl.BlockSpec((1000, D), lambda i: (i, 0)),
    )(partial)


@jax.jit
def kernel(data, edge_index):
    se = edge_index[0]
    de = edge_index[1]
    zeros = jnp.zeros((N, D), jnp.float32)
    partial = _sc_partial(data, se, de, zeros)
    return _combine(partial)
